# baseline (speedup 1.0000x reference)
import functools

import jax
import jax.numpy as jnp
from jax import lax
from jax.experimental import pallas as pl
from jax.experimental.pallas import tpu as pltpu
from jax.experimental.pallas import tpu_sc as plsc

NC, NS, L = 2, 16, 16
NW = NC * NS
B, T, D = 256, 300, 512
ROWS = B * T
RPW = ROWS // NW
TCH = 120
PAIR = 2 * T
JCH = PAIR // TCH
NPAIR = RPW // PAIR
CPW = RPW // TCH

_mesh = plsc.VectorSubcoreMesh(core_axis_name="c", subcore_axis_name="s")


@functools.partial(
    pl.kernel,
    out_type=jax.ShapeDtypeStruct((ROWS, D), jnp.float32),
    mesh=_mesh,
    scratch_types=[
        pltpu.VMEM((CPW, TCH), jnp.int32),
        pltpu.VMEM((TCH, D), jnp.float32),
        pltpu.VMEM((TCH, D), jnp.float32),
        pltpu.SemaphoreType.DMA,
    ],
)
def _embed(idx_hbm, tok_hbm, pos2_hbm, out_hbm, idx_v, pos_v, rows_v, sem):
    wid = lax.axis_index("s") * NC + lax.axis_index("c")
    base = wid * RPW
    pltpu.sync_copy(idx_hbm.at[wid], idx_v)

    for j in range(JCH):
        pltpu.sync_copy(pos2_hbm.at[pl.ds(j * TCH, TCH), :], pos_v)

        def pair_body(p, _, j=j):
            k = p * JCH + j
            pltpu.async_copy(tok_hbm.at[idx_v.at[k]], rows_v, sem).wait()

            def add_row(r, _):
                for c in range(D // L):
                    sl = pl.ds(c * L, L)
                    rows_v[r, sl] = rows_v[r, sl] + pos_v[r, sl]
                return 0

            lax.fori_loop(0, TCH, add_row, 0)
            off = pl.multiple_of(base + k * TCH, 8)
            pltpu.sync_copy(rows_v, out_hbm.at[pl.ds(off, TCH), :])
            return 0

        lax.fori_loop(0, NPAIR, pair_body, 0)


def kernel(idx, token_table, pos_table):
    idx2 = idx.reshape(NW, CPW, TCH)
    pos2 = jnp.concatenate([pos_table, pos_table], axis=0)
    out = _embed(idx2, token_table, pos2)
    return out.reshape(B, T, D)
